# Initial kernel scaffold; baseline (speedup 1.0000x reference)
import jax
import jax.numpy as jnp
from jax.experimental import pallas as pl


def kernel(x, W_parent, b_parent, W_child0, b_child0, W_child1, b_child1, device):
    raise NotImplementedError("write your pallas kernel here")


# speedup vs baseline: 13.6455x; 13.6455x over previous
import functools

import jax
import jax.numpy as jnp
from jax.experimental import pallas as pl
from jax.experimental.pallas import tpu as pltpu

_B, _D, _P, _C = 4096, 2048, 16, 128
_TB = 1024


def _body(x_ref, xb_ref, wp_ref, bp_ref, w0_ref, b0_ref, w1_ref, b1_ref,
          logits_ref, out0_ref, out1_ref, pc_ref):
    p = pl.program_id(1)

    @pl.when(p == 0)
    def _compute_routing():
        logits = jax.lax.dot_general(
            xb_ref[...], wp_ref[...].astype(jnp.bfloat16),
            (((1,), (1,)), ((), ())),
            preferred_element_type=jnp.float32) + bp_ref[...]
        logits_ref[...] = logits
        pc_ref[...] = jnp.argmax(logits, axis=1, keepdims=True).astype(jnp.int32)

    mask = pc_ref[...] == p
    xb = xb_ref[...]
    y0 = jax.lax.dot_general(
        xb, w0_ref[0], (((1,), (1,)), ((), ())),
        preferred_element_type=jnp.float32) + b0_ref[0, 0]
    y1 = jax.lax.dot_general(
        xb, w1_ref[0], (((1,), (1,)), ((), ())),
        preferred_element_type=jnp.float32) + b1_ref[0, 0]

    @pl.when(p == 0)
    def _init():
        out0_ref[...] = jnp.where(mask, y0, 0.0)
        out1_ref[...] = jnp.where(mask, y1, 0.0)

    @pl.when(p > 0)
    def _merge():
        out0_ref[...] = jnp.where(mask, y0, out0_ref[...])
        out1_ref[...] = jnp.where(mask, y1, out1_ref[...])


@jax.jit
def kernel(x, W_parent, b_parent, W_child0, b_child0, W_child1, b_child1,
           device):
    del device
    nb = _B // _TB
    xb16 = x.astype(jnp.bfloat16)
    w0b = W_child0.astype(jnp.bfloat16)
    w1b = W_child1.astype(jnp.bfloat16)
    grid = (nb, _P)
    out_shapes = (
        jax.ShapeDtypeStruct((_B, _P), jnp.float32),
        jax.ShapeDtypeStruct((_B, _C), jnp.float32),
        jax.ShapeDtypeStruct((_B, _C), jnp.float32),
    )
    in_specs = [
        pl.BlockSpec((_TB, _D), lambda i, j: (i, 0)),
        pl.BlockSpec((_TB, _D), lambda i, j: (i, 0)),
        pl.BlockSpec((_P, _D), lambda i, j: (0, 0)),
        pl.BlockSpec((1, _P), lambda i, j: (0, 0)),
        pl.BlockSpec((1, _C, _D), lambda i, j: (j, 0, 0)),
        pl.BlockSpec((1, 1, _C), lambda i, j: (j, 0, 0)),
        pl.BlockSpec((1, _C, _D), lambda i, j: (j, 0, 0)),
        pl.BlockSpec((1, 1, _C), lambda i, j: (j, 0, 0)),
    ]
    out_specs = (
        pl.BlockSpec((_TB, _P), lambda i, j: (i, 0)),
        pl.BlockSpec((_TB, _C), lambda i, j: (i, 0)),
        pl.BlockSpec((_TB, _C), lambda i, j: (i, 0)),
    )
    return pl.pallas_call(
        _body,
        grid=grid,
        in_specs=in_specs,
        out_specs=out_specs,
        out_shape=out_shapes,
        scratch_shapes=[pltpu.VMEM((_TB, 1), jnp.int32)],
        compiler_params=pltpu.CompilerParams(
            dimension_semantics=("arbitrary", "arbitrary")),
    )(x, xb16, W_parent, b_parent.reshape(1, _P), w0b,
      b_child0.reshape(_P, 1, _C), w1b, b_child1.reshape(_P, 1, _C))


# speedup vs baseline: 15.1035x; 1.1068x over previous
import functools

import jax
import jax.numpy as jnp
from jax.experimental import pallas as pl
from jax.experimental.pallas import tpu as pltpu

_B, _D, _P, _C = 4096, 2048, 16, 128
_TB = 1024


def _body(x_ref, wp_ref, bp_ref, w0_ref, b0_ref, w1_ref, b1_ref,
          logits_ref, out0_ref, out1_ref, pc_ref, xb_ref):
    p = pl.program_id(1)

    @pl.when(p == 0)
    def _compute_routing():
        xb = x_ref[...].astype(jnp.bfloat16)
        xb_ref[...] = xb
        logits = jax.lax.dot_general(
            xb, wp_ref[...].astype(jnp.bfloat16),
            (((1,), (1,)), ((), ())),
            preferred_element_type=jnp.float32) + bp_ref[...]
        logits_ref[...] = logits
        pc_ref[...] = jnp.argmax(logits, axis=1, keepdims=True).astype(jnp.int32)

    mask = pc_ref[...] == p
    xb = xb_ref[...]
    y0 = jax.lax.dot_general(
        xb, w0_ref[0], (((1,), (1,)), ((), ())),
        preferred_element_type=jnp.float32) + b0_ref[0, 0]
    y1 = jax.lax.dot_general(
        xb, w1_ref[0], (((1,), (1,)), ((), ())),
        preferred_element_type=jnp.float32) + b1_ref[0, 0]

    @pl.when(p == 0)
    def _init():
        out0_ref[...] = jnp.where(mask, y0, 0.0)
        out1_ref[...] = jnp.where(mask, y1, 0.0)

    @pl.when(p > 0)
    def _merge():
        out0_ref[...] = jnp.where(mask, y0, out0_ref[...])
        out1_ref[...] = jnp.where(mask, y1, out1_ref[...])


@jax.jit
def kernel(x, W_parent, b_parent, W_child0, b_child0, W_child1, b_child1,
           device):
    del device
    nb = _B // _TB
    w0b = W_child0.astype(jnp.bfloat16)
    w1b = W_child1.astype(jnp.bfloat16)
    grid = (nb, _P)
    out_shapes = (
        jax.ShapeDtypeStruct((_B, _P), jnp.float32),
        jax.ShapeDtypeStruct((_B, _C), jnp.float32),
        jax.ShapeDtypeStruct((_B, _C), jnp.float32),
    )
    in_specs = [
        pl.BlockSpec((_TB, _D), lambda i, j: (i, 0)),
        pl.BlockSpec((_P, _D), lambda i, j: (0, 0)),
        pl.BlockSpec((1, _P), lambda i, j: (0, 0)),
        pl.BlockSpec((1, _C, _D), lambda i, j: (j, 0, 0)),
        pl.BlockSpec((1, 1, _C), lambda i, j: (j, 0, 0)),
        pl.BlockSpec((1, _C, _D), lambda i, j: (j, 0, 0)),
        pl.BlockSpec((1, 1, _C), lambda i, j: (j, 0, 0)),
    ]
    out_specs = (
        pl.BlockSpec((_TB, _P), lambda i, j: (i, 0)),
        pl.BlockSpec((_TB, _C), lambda i, j: (i, 0)),
        pl.BlockSpec((_TB, _C), lambda i, j: (i, 0)),
    )
    return pl.pallas_call(
        _body,
        grid=grid,
        in_specs=in_specs,
        out_specs=out_specs,
        out_shape=out_shapes,
        scratch_shapes=[
            pltpu.VMEM((_TB, 1), jnp.int32),
            pltpu.VMEM((_TB, _D), jnp.bfloat16),
        ],
        compiler_params=pltpu.CompilerParams(
            dimension_semantics=("arbitrary", "arbitrary")),
    )(x, W_parent, b_parent.reshape(1, _P), w0b,
      b_child0.reshape(_P, 1, _C), w1b, b_child1.reshape(_P, 1, _C))


# speedup vs baseline: 21.2175x; 1.4048x over previous
import functools

import jax
import jax.numpy as jnp
from jax.experimental import pallas as pl
from jax.experimental.pallas import tpu as pltpu

_B, _D, _P, _C = 4096, 2048, 16, 128
_TB = 1024


def _body(x_ref, wp_ref, bp_ref, wc_ref, bc_ref,
          logits_ref, out0_ref, out1_ref, pc_ref, xb_ref):
    p = pl.program_id(1)

    @pl.when(p == 0)
    def _compute_routing():
        xb = x_ref[...].astype(jnp.bfloat16)
        xb_ref[...] = xb
        logits = jax.lax.dot_general(
            xb, wp_ref[...].astype(jnp.bfloat16),
            (((1,), (1,)), ((), ())),
            preferred_element_type=jnp.float32) + bp_ref[...]
        logits_ref[...] = logits
        pc_ref[...] = jnp.argmax(logits, axis=1, keepdims=True).astype(jnp.int32)

    mask = pc_ref[...] == p
    y = jax.lax.dot_general(
        xb_ref[...], wc_ref[0], (((1,), (1,)), ((), ())),
        preferred_element_type=jnp.float32) + bc_ref[0, 0]

    @pl.when(p == 0)
    def _init():
        out0_ref[...] = jnp.where(mask, y[:, :_C], 0.0)
        out1_ref[...] = jnp.where(mask, y[:, _C:], 0.0)

    @pl.when(p > 0)
    def _merge():
        out0_ref[...] = jnp.where(mask, y[:, :_C], out0_ref[...])
        out1_ref[...] = jnp.where(mask, y[:, _C:], out1_ref[...])


@jax.jit
def kernel(x, W_parent, b_parent, W_child0, b_child0, W_child1, b_child1,
           device):
    del device
    nb = _B // _TB
    wc = jnp.concatenate([W_child0, W_child1], axis=1).astype(jnp.bfloat16)
    bc = jnp.concatenate([b_child0, b_child1], axis=1).reshape(_P, 1, 2 * _C)
    grid = (nb, _P)
    out_shapes = (
        jax.ShapeDtypeStruct((_B, _P), jnp.float32),
        jax.ShapeDtypeStruct((_B, _C), jnp.float32),
        jax.ShapeDtypeStruct((_B, _C), jnp.float32),
    )
    in_specs = [
        pl.BlockSpec((_TB, _D), lambda i, j: (i, 0)),
        pl.BlockSpec((_P, _D), lambda i, j: (0, 0)),
        pl.BlockSpec((1, _P), lambda i, j: (0, 0)),
        pl.BlockSpec((1, 2 * _C, _D), lambda i, j: (j, 0, 0)),
        pl.BlockSpec((1, 1, 2 * _C), lambda i, j: (j, 0, 0)),
    ]
    out_specs = (
        pl.BlockSpec((_TB, _P), lambda i, j: (i, 0)),
        pl.BlockSpec((_TB, _C), lambda i, j: (i, 0)),
        pl.BlockSpec((_TB, _C), lambda i, j: (i, 0)),
    )
    return pl.pallas_call(
        _body,
        grid=grid,
        in_specs=in_specs,
        out_specs=out_specs,
        out_shape=out_shapes,
        scratch_shapes=[
            pltpu.VMEM((_TB, 1), jnp.int32),
            pltpu.VMEM((_TB, _D), jnp.bfloat16),
        ],
        compiler_params=pltpu.CompilerParams(
            dimension_semantics=("arbitrary", "arbitrary")),
    )(x, W_parent, b_parent.reshape(1, _P), wc, bc)
